# Initial kernel scaffold; baseline (speedup 1.0000x reference)
import jax
import jax.numpy as jnp
from jax.experimental import pallas as pl


def kernel(x, token_embeds):
    raise NotImplementedError("write your pallas kernel here")


# speedup vs baseline: 1.0037x; 1.0037x over previous
import jax
import jax.numpy as jnp
from jax import lax
from jax.experimental import pallas as pl

_BLOCK_ROWS = 2000


def _prompt_block_kernel(x_ref, t_ref, o_ref):
    x_blk = x_ref[...]
    t = t_ref[...]
    logits = lax.dot_general(
        x_blk, t, (((1,), (1,)), ((), ())),
        preferred_element_type=jnp.float32)
    m = jnp.max(logits, axis=1, keepdims=True)
    e = jnp.exp(logits - m)
    attn = e / jnp.sum(e, axis=1, keepdims=True)
    prompt = lax.dot_general(
        attn, t, (((1,), (0,)), ((), ())),
        preferred_element_type=jnp.float32)
    o_ref[...] = x_blk + prompt


def kernel(x, token_embeds):
    n, d = x.shape
    t_num = token_embeds.shape[0]
    bn = _BLOCK_ROWS
    grid = (pl.cdiv(n, bn),)
    return pl.pallas_call(
        _prompt_block_kernel,
        grid=grid,
        in_specs=[
            pl.BlockSpec((bn, d), lambda i: (i, 0)),
            pl.BlockSpec((t_num, d), lambda i: (0, 0)),
        ],
        out_specs=pl.BlockSpec((bn, d), lambda i: (i, 0)),
        out_shape=jax.ShapeDtypeStruct((n, d), x.dtype),
    )(x, token_embeds)


# speedup vs baseline: 1.3250x; 1.3201x over previous
import jax
import jax.numpy as jnp
from jax import lax
from jax.experimental import pallas as pl

_BLOCK_ROWS = 4000


def _prompt_block_kernel(x_ref, t_ref, o_ref):
    x_blk = x_ref[...]
    t = t_ref[...]
    logits = lax.dot_general(
        x_blk, t, (((1,), (1,)), ((), ())),
        preferred_element_type=jnp.float32)
    m = jnp.max(logits, axis=1, keepdims=True)
    e = jnp.exp(logits - m)
    attn = e / jnp.sum(e, axis=1, keepdims=True)
    prompt = lax.dot_general(
        attn, t, (((1,), (0,)), ((), ())),
        preferred_element_type=jnp.float32)
    o_ref[...] = x_blk + prompt


def kernel(x, token_embeds):
    n, d = x.shape
    t_num = token_embeds.shape[0]
    bn = _BLOCK_ROWS
    grid = (pl.cdiv(n, bn),)
    return pl.pallas_call(
        _prompt_block_kernel,
        grid=grid,
        in_specs=[
            pl.BlockSpec((bn, d), lambda i: (i, 0)),
            pl.BlockSpec((t_num, d), lambda i: (0, 0)),
        ],
        out_specs=pl.BlockSpec((bn, d), lambda i: (i, 0)),
        out_shape=jax.ShapeDtypeStruct((n, d), x.dtype),
    )(x, token_embeds)


# speedup vs baseline: 1.6404x; 1.2381x over previous
import jax
import jax.numpy as jnp
from jax import lax
from jax.experimental import pallas as pl

_BLOCK_ROWS = 10000


def _prompt_block_kernel(x_ref, t_ref, o_ref):
    x_blk = x_ref[...]
    t = t_ref[...]
    logits = lax.dot_general(
        x_blk, t, (((1,), (1,)), ((), ())),
        preferred_element_type=jnp.float32)
    m = jnp.max(logits, axis=1, keepdims=True)
    e = jnp.exp(logits - m)
    attn = e / jnp.sum(e, axis=1, keepdims=True)
    prompt = lax.dot_general(
        attn, t, (((1,), (0,)), ((), ())),
        preferred_element_type=jnp.float32)
    o_ref[...] = x_blk + prompt


def kernel(x, token_embeds):
    n, d = x.shape
    t_num = token_embeds.shape[0]
    bn = _BLOCK_ROWS
    grid = (pl.cdiv(n, bn),)
    return pl.pallas_call(
        _prompt_block_kernel,
        grid=grid,
        in_specs=[
            pl.BlockSpec((bn, d), lambda i: (i, 0)),
            pl.BlockSpec((t_num, d), lambda i: (0, 0)),
        ],
        out_specs=pl.BlockSpec((bn, d), lambda i: (i, 0)),
        out_shape=jax.ShapeDtypeStruct((n, d), x.dtype),
    )(x, token_embeds)


# speedup vs baseline: 1.6505x; 1.0061x over previous
import jax
import jax.numpy as jnp
from jax import lax
from jax.experimental import pallas as pl

_BLOCK_ROWS = 20000


def _prompt_block_kernel(x_ref, t_ref, o_ref):
    x_blk = x_ref[...]
    t = t_ref[...]
    logits = lax.dot_general(
        x_blk, t, (((1,), (1,)), ((), ())),
        preferred_element_type=jnp.float32)
    m = jnp.max(logits, axis=1, keepdims=True)
    e = jnp.exp(logits - m)
    attn = e / jnp.sum(e, axis=1, keepdims=True)
    prompt = lax.dot_general(
        attn, t, (((1,), (0,)), ((), ())),
        preferred_element_type=jnp.float32)
    o_ref[...] = x_blk + prompt


def kernel(x, token_embeds):
    n, d = x.shape
    t_num = token_embeds.shape[0]
    bn = _BLOCK_ROWS
    grid = (pl.cdiv(n, bn),)
    return pl.pallas_call(
        _prompt_block_kernel,
        grid=grid,
        in_specs=[
            pl.BlockSpec((bn, d), lambda i: (i, 0)),
            pl.BlockSpec((t_num, d), lambda i: (0, 0)),
        ],
        out_specs=pl.BlockSpec((bn, d), lambda i: (i, 0)),
        out_shape=jax.ShapeDtypeStruct((n, d), x.dtype),
    )(x, token_embeds)


# speedup vs baseline: 1.6623x; 1.0072x over previous
import jax
import jax.numpy as jnp
from jax import lax
from jax.experimental import pallas as pl
from jax.experimental.pallas import tpu as pltpu

_BLOCK_ROWS = 25000


def _prompt_block_kernel(x_ref, t_ref, o_ref):
    x_blk = x_ref[...]
    t = t_ref[...]
    logits = lax.dot_general(
        x_blk, t, (((1,), (1,)), ((), ())),
        preferred_element_type=jnp.float32)
    m = jnp.max(logits, axis=1, keepdims=True)
    e = jnp.exp(logits - m)
    attn = e / jnp.sum(e, axis=1, keepdims=True)
    prompt = lax.dot_general(
        attn, t, (((1,), (0,)), ((), ())),
        preferred_element_type=jnp.float32)
    o_ref[...] = x_blk + prompt


def kernel(x, token_embeds):
    n, d = x.shape
    t_num = token_embeds.shape[0]
    bn = _BLOCK_ROWS
    grid = (pl.cdiv(n, bn),)
    return pl.pallas_call(
        _prompt_block_kernel,
        grid=grid,
        in_specs=[
            pl.BlockSpec((bn, d), lambda i: (i, 0)),
            pl.BlockSpec((t_num, d), lambda i: (0, 0)),
        ],
        out_specs=pl.BlockSpec((bn, d), lambda i: (i, 0)),
        out_shape=jax.ShapeDtypeStruct((n, d), x.dtype),
        compiler_params=pltpu.CompilerParams(
            dimension_semantics=("parallel",)),
    )(x, token_embeds)
